# baseline (speedup 1.0000x reference)
import functools

import jax
import jax.numpy as jnp
from jax import lax
from jax.experimental import pallas as pl
from jax.experimental.pallas import tpu as pltpu
from jax.experimental.pallas import tpu_sc as plsc

VOCAB = 1000000
EMBED_DIM = 64
BATCH = 4096
SEQ_LEN = 200

N = BATCH * SEQ_LEN
GRP = 128
GROUPS_PER_CHUNK = 4
CHUNK = GRP * GROUPS_PER_CHUNK


def _make_sc_gather():
    info = plsc.get_sparse_core_info()
    nc, ns = info.num_cores, info.num_subcores
    nw = nc * ns
    per_w = N // nw
    groups_per_w = per_w // GRP
    chunks_per_w = per_w // CHUNK

    mesh = plsc.VectorSubcoreMesh(core_axis_name="c", subcore_axis_name="s")

    @functools.partial(
        pl.kernel,
        mesh=mesh,
        compiler_params=pltpu.CompilerParams(use_tc_tiling_on_sc=False),
        out_type=jax.ShapeDtypeStruct((N, EMBED_DIM), jnp.float32),
        scratch_types=[
            pltpu.VMEM((groups_per_w, GRP), jnp.int32),
            pltpu.VMEM((CHUNK, EMBED_DIM), jnp.float32),
            pltpu.SemaphoreType.DMA,
        ],
    )
    def k(idx_hbm, table_hbm, out_hbm, idx_v, rows_v, sem):
        wid = lax.axis_index("s") * nc + lax.axis_index("c")
        base = wid * per_w
        pltpu.sync_copy(idx_hbm.at[wid], idx_v)

        def chunk_body(c, _):
            handles = []
            for j in range(GROUPS_PER_CHUNK):
                g = c * GROUPS_PER_CHUNK + j
                handles.append(
                    pltpu.async_copy(
                        table_hbm.at[idx_v.at[g]],
                        rows_v.at[pl.ds(j * GRP, GRP)],
                        sem,
                    )
                )
            for h in handles:
                h.wait()
            pltpu.sync_copy(rows_v, out_hbm.at[pl.ds(base + c * CHUNK, CHUNK)])
            return _

        lax.fori_loop(0, chunks_per_w, chunk_body, 0)

    return k, nw, per_w


def kernel(batch, table):
    k, nw, per_w = _make_sc_gather()
    idx = batch.reshape(nw, per_w // GRP, GRP)
    out = k(idx, table)
    return out.reshape(BATCH, SEQ_LEN, EMBED_DIM)


# speedup vs baseline: 1.0239x; 1.0239x over previous
import functools

import jax
import jax.numpy as jnp
from jax import lax
from jax.experimental import pallas as pl
from jax.experimental.pallas import tpu as pltpu
from jax.experimental.pallas import tpu_sc as plsc

VOCAB = 1000000
EMBED_DIM = 64
BATCH = 4096
SEQ_LEN = 200

N = BATCH * SEQ_LEN
GRP = 128
GROUPS_PER_CHUNK = 4
CHUNK = GRP * GROUPS_PER_CHUNK


def _make_sc_gather():
    info = plsc.get_sparse_core_info()
    nc, ns = info.num_cores, info.num_subcores
    nw = nc * ns
    per_w = N // nw
    groups_per_w = per_w // GRP
    chunks_per_w = per_w // CHUNK

    npairs = chunks_per_w // 2

    mesh = plsc.VectorSubcoreMesh(core_axis_name="c", subcore_axis_name="s")

    @functools.partial(
        pl.kernel,
        mesh=mesh,
        compiler_params=pltpu.CompilerParams(use_tc_tiling_on_sc=False),
        out_type=jax.ShapeDtypeStruct((N, EMBED_DIM), jnp.float32),
        scratch_types=[
            pltpu.VMEM((groups_per_w, GRP), jnp.int32),
            pltpu.VMEM((2, CHUNK, EMBED_DIM), jnp.float32),
            pltpu.SemaphoreType.DMA,
            pltpu.SemaphoreType.DMA,
            pltpu.SemaphoreType.DMA,
            pltpu.SemaphoreType.DMA,
        ],
    )
    def k(idx_hbm, table_hbm, out_hbm, idx_v, rows_v, g0, g1, w0, w1):
        wid = lax.axis_index("s") * nc + lax.axis_index("c")
        base = wid * per_w
        gsem = (g0, g1)
        wsem = (w0, w1)
        pltpu.sync_copy(idx_hbm.at[wid], idx_v)

        def issue_gather(c, b):
            for j in range(GROUPS_PER_CHUNK):
                g = c * GROUPS_PER_CHUNK + j
                pltpu.async_copy(
                    table_hbm.at[idx_v.at[g]],
                    rows_v.at[b].at[pl.ds(j * GRP, GRP)],
                    gsem[b],
                )

        def wait_gather(b):
            pltpu.make_async_copy(
                out_hbm.at[pl.ds(base, CHUNK)], rows_v.at[b], gsem[b]
            ).wait()

        def issue_wb(c, b):
            pltpu.async_copy(
                rows_v.at[b], out_hbm.at[pl.ds(base + c * CHUNK, CHUNK)], wsem[b]
            )

        def wait_wb(b):
            pltpu.make_async_copy(
                rows_v.at[b], out_hbm.at[pl.ds(base, CHUNK)], wsem[b]
            ).wait()

        issue_gather(0, 0)

        def body(p, carry):
            for b in (0, 1):
                c = 2 * p + b
                if b == 0:
                    @pl.when(p > 0)
                    def _():
                        wait_wb(1)
                    issue_gather(c + 1, 1)
                else:
                    wait_wb(0)

                    @pl.when(p < npairs - 1)
                    def _():
                        issue_gather(c + 1, 0)
                wait_gather(b)
                issue_wb(c, b)
            return carry

        lax.fori_loop(0, npairs, body, 0)
        wait_wb(1)

    return k, nw, per_w


def kernel(batch, table):
    k, nw, per_w = _make_sc_gather()
    idx = batch.reshape(nw, per_w // GRP, GRP)
    out = k(idx, table)
    return out.reshape(BATCH, SEQ_LEN, EMBED_DIM)
